# Initial kernel scaffold; baseline (speedup 1.0000x reference)
import jax
import jax.numpy as jnp
from jax.experimental import pallas as pl


def kernel(x, edge_index, W1, b1, W2, b2):
    raise NotImplementedError("write your pallas kernel here")


# speedup vs baseline: 2.7934x; 2.7934x over previous
import functools

import jax
import jax.numpy as jnp
from jax.experimental import pallas as pl
from jax.experimental.pallas import tpu as pltpu

N_NODES = 10000
BLK = 2000


def _mm_body(x_ref, w_ref, o_ref):
    o_ref[...] = jnp.dot(x_ref[...], w_ref[...],
                         preferred_element_type=jnp.float32)


def _matmul(x, w):
    m, k = x.shape
    _, n = w.shape
    grid = (m // BLK,)
    return pl.pallas_call(
        _mm_body,
        grid=grid,
        in_specs=[
            pl.BlockSpec((BLK, k), lambda i: (i, 0)),
            pl.BlockSpec((k, n), lambda i: (0, 0)),
        ],
        out_specs=pl.BlockSpec((BLK, n), lambda i: (i, 0)),
        out_shape=jax.ShapeDtypeStruct((m, n), jnp.float32),
    )(x, w)


def kernel(x, edge_index, W1, b1, W2, b2):
    n = x.shape[0]
    src = edge_index[0].astype(jnp.int32)
    dst = edge_index[1].astype(jnp.int32)

    deg = jnp.ones((n,), jnp.float32).at[dst].add(1.0)
    dis = jax.lax.rsqrt(deg)

    y = _matmul(x, W1)
    z = dis[:, None] * y
    agg = jnp.zeros((n, 128), jnp.float32).at[dst].add(z[src])
    h = jax.nn.relu(dis[:, None] * (agg + z) + b1)

    g = _matmul(h, jnp.pad(W2, ((0, 0), (0, 126))))
    zg = dis[:, None] * g
    agg2 = jnp.zeros((n, 128), jnp.float32).at[dst].add(zg[src])
    out = dis[:, None] * (agg2 + zg)
    out = out[:, :2] + b2
    return out


# speedup vs baseline: 22.8915x; 8.1948x over previous
import functools

import jax
import jax.numpy as jnp
from jax import lax
from jax.experimental import pallas as pl
from jax.experimental.pallas import tpu as pltpu
from jax.experimental.pallas import tpu_sc as plsc

FDIM = 128
CHUNK = 64

_NC = 2
_NS = 16
_NW = _NC * _NS


def _spmm_body(nch, rpt, z_hbm, srcr, dstr, out_hbm,
               src_v, dst_v, bufa, bufb, acc,
               sga, sgb, ssa, ssb):
    c = lax.axis_index("c")
    s = lax.axis_index("s")
    wid = c * _NS + s

    pltpu.sync_copy(srcr.at[wid], src_v)
    pltpu.sync_copy(dstr.at[wid], dst_v)

    for r in range(rpt // CHUNK):
        base = s * rpt + r * CHUNK
        pltpu.sync_copy(z_hbm.at[pl.ds(base, CHUNK)], bufa)
        pltpu.sync_copy(bufa, acc.at[pl.ds(base, CHUNK)])
    plsc.subcore_barrier()

    def body(i, carry):
        j0 = 2 * i
        j1 = 2 * i + 1
        ga = pltpu.async_copy(z_hbm.at[src_v.at[j0]], bufa, sga)
        gb = pltpu.async_copy(z_hbm.at[src_v.at[j1]], bufb, sgb)
        ga.wait()
        sa = pltpu.async_copy(bufa, acc.at[dst_v.at[j0]], ssa, add=True)
        gb.wait()
        sb = pltpu.async_copy(bufb, acc.at[dst_v.at[j1]], ssb, add=True)
        sa.wait()
        sb.wait()
        return carry

    lax.fori_loop(0, nch // 2, body, 0)
    plsc.subcore_barrier()

    for r in range(rpt // CHUNK):
        base = s * rpt + r * CHUNK
        pltpu.sync_copy(acc.at[pl.ds(base, CHUNK)], bufa)
        pltpu.sync_copy(bufa, out_hbm.at[c, pl.ds(base, CHUNK)])


@functools.lru_cache(maxsize=None)
def _make_spmm(npad, nch, width):
    rpt = npad // _NS
    mesh = plsc.VectorSubcoreMesh(core_axis_name="c", subcore_axis_name="s")
    return pl.kernel(
        functools.partial(_spmm_body, nch, rpt),
        out_type=jax.ShapeDtypeStruct((_NC, npad, width), jnp.float32),
        mesh=mesh,
        compiler_params=pltpu.CompilerParams(use_tc_tiling_on_sc=False),
        scratch_types=[
            pltpu.VMEM((nch, CHUNK), jnp.int32),
            pltpu.VMEM((nch, CHUNK), jnp.int32),
            pltpu.VMEM((CHUNK, width), jnp.float32),
            pltpu.VMEM((CHUNK, width), jnp.float32),
            pltpu.VMEM_SHARED((npad, width), jnp.float32),
            pltpu.SemaphoreType.DMA,
            pltpu.SemaphoreType.DMA,
            pltpu.SemaphoreType.DMA,
            pltpu.SemaphoreType.DMA,
        ],
    )


def _spmm(z, src3, dst3):
    return _make_spmm(z.shape[0], src3.shape[1], z.shape[1])(z, src3, dst3)


def _mm_scaled_body(x_ref, w_ref, d_ref, o_ref):
    o_ref[...] = d_ref[...] * jnp.dot(x_ref[...], w_ref[...],
                                      preferred_element_type=jnp.float32)


def _matmul_scaled(x, w, disb, blk):
    m, k = x.shape
    _, n = w.shape
    return pl.pallas_call(
        _mm_scaled_body,
        grid=(m // blk,),
        in_specs=[
            pl.BlockSpec((blk, k), lambda i: (i, 0)),
            pl.BlockSpec((k, n), lambda i: (0, 0)),
            pl.BlockSpec((blk, n), lambda i: (i, 0)),
        ],
        out_specs=pl.BlockSpec((blk, n), lambda i: (i, 0)),
        out_shape=jax.ShapeDtypeStruct((m, n), jnp.float32),
    )(x, w, disb)


def _combine_mm_body(a_ref, z_ref, d_ref, b_ref, w_ref, o_ref):
    h = d_ref[...] * (a_ref[0] + a_ref[1] - z_ref[...]) + b_ref[...]
    h = jnp.maximum(h, 0.0)
    o_ref[...] = jnp.dot(h, w_ref[...], preferred_element_type=jnp.float32)


def _combine_matmul(acc, z, disb, b1, w2p, blk):
    m, k = z.shape
    return pl.pallas_call(
        _combine_mm_body,
        grid=(m // blk,),
        in_specs=[
            pl.BlockSpec((2, blk, k), lambda i: (0, i, 0)),
            pl.BlockSpec((blk, k), lambda i: (i, 0)),
            pl.BlockSpec((blk, k), lambda i: (i, 0)),
            pl.BlockSpec((1, k), lambda i: (0, 0)),
            pl.BlockSpec((k, k), lambda i: (0, 0)),
        ],
        out_specs=pl.BlockSpec((blk, k), lambda i: (i, 0)),
        out_shape=jax.ShapeDtypeStruct((m, k), jnp.float32),
    )(acc, z, disb, b1, w2p)


def kernel(x, edge_index, W1, b1, W2, b2):
    n = x.shape[0]
    e = edge_index.shape[1]
    npad = ((n + 2047) // 2048) * 2048
    egrain = _NW * CHUNK * 2
    epad = ((e + egrain - 1) // egrain) * egrain
    blk = npad // 5

    ei = edge_index.astype(jnp.int32)
    fill = n + (jnp.arange(epad - e, dtype=jnp.int32) % (npad - n))
    src3 = jnp.concatenate([ei[0], fill]).reshape(_NW, -1, CHUNK)
    dst3 = jnp.concatenate([ei[1], fill]).reshape(_NW, -1, CHUNK)

    ones16 = jnp.ones((npad, 16), jnp.float32)
    dacc = _spmm(ones16, src3, dst3)
    deg = dacc[0, :, 0] + dacc[1, :, 0] - 1.0
    dis = lax.rsqrt(deg)
    disb = jnp.broadcast_to(dis[:, None], (npad, FDIM))

    xp = jnp.pad(x, ((0, npad - n), (0, 0)))

    z = _matmul_scaled(xp, W1, disb, blk)
    acc = _spmm(z, src3, dst3)
    w2p = jnp.pad(W2, ((0, 0), (0, FDIM - W2.shape[1])))
    g = _combine_matmul(acc, z, disb, b1[None, :], w2p, blk)

    zg = dis[:, None] * g[:, :16]
    acc2 = _spmm(zg, src3, dst3)
    out = dis[:, None] * (acc2[0] + acc2[1] - zg)
    return out[:n, :2] + b2


# speedup vs baseline: 32.6404x; 1.4259x over previous
import functools

import jax
import jax.numpy as jnp
from jax import lax
from jax.experimental import pallas as pl
from jax.experimental.pallas import tpu as pltpu
from jax.experimental.pallas import tpu_sc as plsc

FDIM = 128

_NC = 2
_NS = 16
_NW = _NC * _NS


def _spmm_body(nch, rpt, chunk, z_hbm, srcr, dstr, out_hbm,
               src_v, dst_v, b0, b1, b2, acc,
               g0, g1, g2, s0, s1, s2):
    c = lax.axis_index("c")
    s = lax.axis_index("s")
    wid = c * _NS + s
    bufs = (b0, b1, b2)
    gsem = (g0, g1, g2)
    ssem = (s0, s1, s2)

    pltpu.sync_copy(srcr.at[wid], src_v)
    pltpu.sync_copy(dstr.at[wid], dst_v)

    for r in range(rpt // chunk):
        base = s * rpt + r * chunk
        pltpu.sync_copy(z_hbm.at[pl.ds(base, chunk)], b0)
        pltpu.sync_copy(b0, acc.at[pl.ds(base, chunk)])
    plsc.subcore_barrier()

    def gather(j, k):
        return pltpu.async_copy(z_hbm.at[src_v.at[j]], bufs[k], gsem[k])

    def gather_wait(j, k):
        pltpu.make_async_copy(z_hbm.at[src_v.at[j]], bufs[k], gsem[k]).wait()

    def scatter(j, k):
        return pltpu.async_copy(bufs[k], acc.at[dst_v.at[j]], ssem[k],
                                add=True)

    def scatter_wait(j, k):
        pltpu.make_async_copy(bufs[k], acc.at[dst_v.at[j]], ssem[k]).wait()

    gather(0, 0)
    gather(1, 1)
    gather_wait(0, 0)
    scatter(0, 0)
    gather(2, 2)

    def body(i, carry):
        for k, j in ((1, 3 * i + 1), (2, 3 * i + 2), (0, 3 * i + 3)):
            kn = (k + 2) % 3
            gather_wait(j, k)
            scatter(j, k)
            scatter_wait(j - 1, kn)
            gather(j + 2, kn)
        return carry

    lax.fori_loop(0, (nch - 3) // 3, body, 0)
    for j, k in ((nch - 2, 1), (nch - 1, 2)):
        gather_wait(j, k)
        scatter(j, k)
    for j, k in ((nch - 3, 0), (nch - 2, 1), (nch - 1, 2)):
        scatter_wait(j, k)
    plsc.subcore_barrier()

    for r in range(rpt // chunk):
        base = s * rpt + r * chunk
        pltpu.sync_copy(acc.at[pl.ds(base, chunk)], b0)
        pltpu.sync_copy(b0, out_hbm.at[c, pl.ds(base, chunk)])


@functools.lru_cache(maxsize=None)
def _make_spmm(npad, nch, width, chunk):
    rpt = npad // _NS
    mesh = plsc.VectorSubcoreMesh(core_axis_name="c", subcore_axis_name="s")
    sem = pltpu.SemaphoreType.DMA
    return pl.kernel(
        functools.partial(_spmm_body, nch, rpt, chunk),
        out_type=jax.ShapeDtypeStruct((_NC, npad, width), jnp.float32),
        mesh=mesh,
        compiler_params=pltpu.CompilerParams(use_tc_tiling_on_sc=False),
        scratch_types=[
            pltpu.VMEM((nch, chunk), jnp.int32),
            pltpu.VMEM((nch, chunk), jnp.int32),
            pltpu.VMEM((chunk, width), jnp.float32),
            pltpu.VMEM((chunk, width), jnp.float32),
            pltpu.VMEM((chunk, width), jnp.float32),
            pltpu.VMEM_SHARED((npad, width), jnp.float32),
            sem, sem, sem, sem, sem, sem,
        ],
    )


def _spmm(z, src3, dst3):
    return _make_spmm(z.shape[0], src3.shape[1], z.shape[1],
                      src3.shape[2])(z, src3, dst3)


def _mm_scaled_body(x_ref, w_ref, d_ref, o_ref):
    o_ref[...] = d_ref[...] * jnp.dot(x_ref[...], w_ref[...],
                                      preferred_element_type=jnp.float32)


def _matmul_scaled(x, w, disb, blk):
    m, k = x.shape
    _, n = w.shape
    return pl.pallas_call(
        _mm_scaled_body,
        grid=(m // blk,),
        in_specs=[
            pl.BlockSpec((blk, k), lambda i: (i, 0)),
            pl.BlockSpec((k, n), lambda i: (0, 0)),
            pl.BlockSpec((blk, n), lambda i: (i, 0)),
        ],
        out_specs=pl.BlockSpec((blk, n), lambda i: (i, 0)),
        out_shape=jax.ShapeDtypeStruct((m, n), jnp.float32),
    )(x, w, disb)


def _combine_mm_body(a_ref, z_ref, d_ref, b_ref, w_ref, o_ref):
    h = d_ref[...] * (a_ref[0] + a_ref[1] - z_ref[...]) + b_ref[...]
    h = jnp.maximum(h, 0.0)
    o_ref[...] = jnp.dot(h, w_ref[...], preferred_element_type=jnp.float32)


def _combine_matmul(acc, z, disb, b1, w2p, blk):
    m, k = z.shape
    return pl.pallas_call(
        _combine_mm_body,
        grid=(m // blk,),
        in_specs=[
            pl.BlockSpec((2, blk, k), lambda i: (0, i, 0)),
            pl.BlockSpec((blk, k), lambda i: (i, 0)),
            pl.BlockSpec((blk, k), lambda i: (i, 0)),
            pl.BlockSpec((1, k), lambda i: (0, 0)),
            pl.BlockSpec((k, k), lambda i: (0, 0)),
        ],
        out_specs=pl.BlockSpec((blk, k), lambda i: (i, 0)),
        out_shape=jax.ShapeDtypeStruct((m, k), jnp.float32),
    )(acc, z, disb, b1, w2p)


def kernel(x, edge_index, W1, b1, W2, b2):
    n = x.shape[0]
    e = edge_index.shape[1]
    npad = ((n + 2047) // 2048) * 2048
    blk = npad // 5
    ei = edge_index.astype(jnp.int32)

    def edge_layout(chunk):
        grain = _NW * chunk * 3
        epad = ((e + grain - 1) // grain) * grain
        fill = n + (jnp.arange(epad - e, dtype=jnp.int32) % (npad - n))
        src3 = jnp.concatenate([ei[0], fill]).reshape(_NW, -1, chunk)
        dst3 = jnp.concatenate([ei[1], fill]).reshape(_NW, -1, chunk)
        return src3, dst3

    srcw, dstw = edge_layout(64)
    srcn, dstn = edge_layout(128)

    ones16 = jnp.ones((npad, 16), jnp.float32)
    dacc = _spmm(ones16, srcn, dstn)
    deg = dacc[0, :, 0] + dacc[1, :, 0] - 1.0
    dis = lax.rsqrt(deg)
    disb = jnp.broadcast_to(dis[:, None], (npad, FDIM))

    xp = jnp.pad(x, ((0, npad - n), (0, 0)))

    z = _matmul_scaled(xp, W1, disb, blk)
    acc = _spmm(z, srcw, dstw)
    w2p = jnp.pad(W2, ((0, 0), (0, FDIM - W2.shape[1])))
    g = _combine_matmul(acc, z, disb, b1[None, :], w2p, blk)

    zg = dis[:, None] * g[:, :16]
    acc2 = _spmm(zg, srcn, dstn)
    out = dis[:, None] * (acc2[0] + acc2[1] - zg)
    return out[:n, :2] + b2


# speedup vs baseline: 34.1317x; 1.0457x over previous
import functools

import jax
import jax.numpy as jnp
from jax import lax
from jax.experimental import pallas as pl
from jax.experimental.pallas import tpu as pltpu
from jax.experimental.pallas import tpu_sc as plsc

FDIM = 128

_NC = 2
_NS = 16
_NW = _NC * _NS


def _spmm_body(nch, rpt, chunk, z_hbm, srcr, dstr, out_hbm,
               src_v, dst_v, b0, b1, b2, acc,
               g0, g1, g2, s0, s1, s2):
    c = lax.axis_index("c")
    s = lax.axis_index("s")
    wid = c * _NS + s
    bufs = (b0, b1, b2)
    gsem = (g0, g1, g2)
    ssem = (s0, s1, s2)

    pltpu.sync_copy(srcr.at[wid], src_v)
    pltpu.sync_copy(dstr.at[wid], dst_v)

    base = s * rpt
    pltpu.sync_copy(z_hbm.at[pl.ds(base, rpt)], acc.at[pl.ds(base, rpt)])
    plsc.subcore_barrier()

    def gather(j, k):
        return pltpu.async_copy(z_hbm.at[src_v.at[j]], bufs[k], gsem[k])

    def gather_wait(j, k):
        pltpu.make_async_copy(z_hbm.at[src_v.at[j]], bufs[k], gsem[k]).wait()

    def scatter(j, k):
        return pltpu.async_copy(bufs[k], acc.at[dst_v.at[j]], ssem[k],
                                add=True)

    def scatter_wait(j, k):
        pltpu.make_async_copy(bufs[k], acc.at[dst_v.at[j]], ssem[k]).wait()

    gather(0, 0)
    gather(1, 1)
    gather_wait(0, 0)
    scatter(0, 0)
    gather(2, 2)

    def body(i, carry):
        for k, j in ((1, 3 * i + 1), (2, 3 * i + 2), (0, 3 * i + 3)):
            kn = (k + 2) % 3
            gather_wait(j, k)
            scatter(j, k)
            scatter_wait(j - 1, kn)
            gather(j + 2, kn)
        return carry

    lax.fori_loop(0, (nch - 3) // 3, body, 0)
    for j, k in ((nch - 2, 1), (nch - 1, 2)):
        gather_wait(j, k)
        scatter(j, k)
    for j, k in ((nch - 3, 0), (nch - 2, 1), (nch - 1, 2)):
        scatter_wait(j, k)
    plsc.subcore_barrier()

    pltpu.sync_copy(acc.at[pl.ds(base, rpt)], out_hbm.at[c, pl.ds(base, rpt)])


@functools.lru_cache(maxsize=None)
def _make_spmm(npad, nch, width, chunk):
    rpt = npad // _NS
    mesh = plsc.VectorSubcoreMesh(core_axis_name="c", subcore_axis_name="s")
    sem = pltpu.SemaphoreType.DMA
    return pl.kernel(
        functools.partial(_spmm_body, nch, rpt, chunk),
        out_type=jax.ShapeDtypeStruct((_NC, npad, width), jnp.float32),
        mesh=mesh,
        compiler_params=pltpu.CompilerParams(use_tc_tiling_on_sc=False),
        scratch_types=[
            pltpu.VMEM((nch, chunk), jnp.int32),
            pltpu.VMEM((nch, chunk), jnp.int32),
            pltpu.VMEM((chunk, width), jnp.float32),
            pltpu.VMEM((chunk, width), jnp.float32),
            pltpu.VMEM((chunk, width), jnp.float32),
            pltpu.VMEM_SHARED((npad, width), jnp.float32),
            sem, sem, sem, sem, sem, sem,
        ],
    )


def _spmm(z, src3, dst3):
    return _make_spmm(z.shape[0], src3.shape[1], z.shape[1],
                      src3.shape[2])(z, src3, dst3)


def _mm_body(x_ref, w_ref, o_ref):
    o_ref[...] = jnp.dot(x_ref[...], w_ref[...],
                         preferred_element_type=jnp.float32)


def _matmul(x, w, blk):
    m, k = x.shape
    _, n = w.shape
    return pl.pallas_call(
        _mm_body,
        grid=(m // blk,),
        in_specs=[
            pl.BlockSpec((blk, k), lambda i: (i, 0)),
            pl.BlockSpec((k, n), lambda i: (0, 0)),
        ],
        out_specs=pl.BlockSpec((blk, n), lambda i: (i, 0)),
        out_shape=jax.ShapeDtypeStruct((m, n), jnp.float32),
    )(x, w)


def _combine_mm_body(a_ref, z_ref, d_ref, b_ref, w_ref, o_ref):
    h = d_ref[...] * (a_ref[0] + a_ref[1] - z_ref[...]) + b_ref[...]
    h = jnp.maximum(h, 0.0)
    o_ref[...] = jnp.dot(h, w_ref[...], preferred_element_type=jnp.float32)


def _combine_matmul(acc, z, disb, b1, w2p, blk):
    m, k = z.shape
    return pl.pallas_call(
        _combine_mm_body,
        grid=(m // blk,),
        in_specs=[
            pl.BlockSpec((2, blk, k), lambda i: (0, i, 0)),
            pl.BlockSpec((blk, k), lambda i: (i, 0)),
            pl.BlockSpec((blk, k), lambda i: (i, 0)),
            pl.BlockSpec((1, k), lambda i: (0, 0)),
            pl.BlockSpec((k, k), lambda i: (0, 0)),
        ],
        out_specs=pl.BlockSpec((blk, k), lambda i: (i, 0)),
        out_shape=jax.ShapeDtypeStruct((m, k), jnp.float32),
    )(acc, z, disb, b1, w2p)


def kernel(x, edge_index, W1, b1, W2, b2):
    n = x.shape[0]
    e = edge_index.shape[1]
    npad = ((n + 2047) // 2048) * 2048
    blk = npad // 5
    ei = edge_index.astype(jnp.int32)

    def edge_layout(chunk):
        grain = _NW * chunk * 3
        epad = ((e + grain - 1) // grain) * grain
        fill = n + (jnp.arange(epad - e, dtype=jnp.int32) % (npad - n))
        src3 = jnp.concatenate([ei[0], fill]).reshape(_NW, -1, chunk)
        dst3 = jnp.concatenate([ei[1], fill]).reshape(_NW, -1, chunk)
        return src3, dst3

    srcw, dstw = edge_layout(64)
    srcn, dstn = edge_layout(128)

    ones16 = jnp.ones((npad, 16), jnp.float32)
    dacc = _spmm(ones16, srcn, dstn)
    xp = jnp.pad(x, ((0, npad - n), (0, 0)))
    y = _matmul(xp, W1, blk)
    deg = dacc[0, :, 0] + dacc[1, :, 0] - 1.0
    dis = lax.rsqrt(deg)
    disb = jnp.broadcast_to(dis[:, None], (npad, FDIM))

    z = disb * y
    acc = _spmm(z, srcw, dstw)
    w2p = jnp.pad(W2, ((0, 0), (0, FDIM - W2.shape[1])))
    g = _combine_matmul(acc, z, disb, b1[None, :], w2p, blk)

    zg = dis[:, None] * g[:, :16]
    acc2 = _spmm(zg, srcn, dstn)
    out = dis[:, None] * (acc2[0] + acc2[1] - zg)
    return out[:n, :2] + b2


# speedup vs baseline: 35.0540x; 1.0270x over previous
import functools

import jax
import jax.numpy as jnp
from jax import lax
from jax.experimental import pallas as pl
from jax.experimental.pallas import tpu as pltpu
from jax.experimental.pallas import tpu_sc as plsc

FDIM = 128

_NC = 2
_NS = 16
_NW = _NC * _NS


def _spmm_body(nch, rpt, chunk, z_hbm, srcr, dstr, out_hbm,
               src_v, dst_v, b0, b1, b2, acc,
               g0, g1, g2, s0, s1, s2):
    c = lax.axis_index("c")
    s = lax.axis_index("s")
    wid = c * _NS + s
    bufs = (b0, b1, b2)
    gsem = (g0, g1, g2)
    ssem = (s0, s1, s2)

    pltpu.sync_copy(srcr.at[wid], src_v)
    pltpu.sync_copy(dstr.at[wid], dst_v)

    base = s * rpt
    pltpu.sync_copy(z_hbm.at[pl.ds(base, rpt)], acc.at[pl.ds(base, rpt)])
    plsc.subcore_barrier()

    def gather(j, k):
        return pltpu.async_copy(z_hbm.at[src_v.at[j]], bufs[k], gsem[k])

    def gather_wait(j, k):
        pltpu.make_async_copy(z_hbm.at[src_v.at[j]], bufs[k], gsem[k]).wait()

    def scatter(j, k):
        return pltpu.async_copy(bufs[k], acc.at[dst_v.at[j]], ssem[k],
                                add=True)

    def scatter_wait(j, k):
        pltpu.make_async_copy(bufs[k], acc.at[dst_v.at[j]], ssem[k]).wait()

    gather(0, 0)
    gather(1, 1)
    gather_wait(0, 0)
    scatter(0, 0)
    gather(2, 2)

    def body(i, carry):
        for k, j in ((1, 3 * i + 1), (2, 3 * i + 2), (0, 3 * i + 3)):
            kn = (k + 2) % 3
            gather_wait(j, k)
            scatter(j, k)
            scatter_wait(j - 1, kn)
            gather(j + 2, kn)
        return carry

    lax.fori_loop(0, (nch - 3) // 3, body, 0)
    for j, k in ((nch - 2, 1), (nch - 1, 2)):
        gather_wait(j, k)
        scatter(j, k)
    for j, k in ((nch - 3, 0), (nch - 2, 1), (nch - 1, 2)):
        scatter_wait(j, k)
    plsc.subcore_barrier()

    pltpu.sync_copy(acc.at[pl.ds(base, rpt)], out_hbm.at[c, pl.ds(base, rpt)])


@functools.lru_cache(maxsize=None)
def _make_spmm(npad, nch, width, chunk):
    rpt = npad // _NS
    mesh = plsc.VectorSubcoreMesh(core_axis_name="c", subcore_axis_name="s")
    sem = pltpu.SemaphoreType.DMA
    return pl.kernel(
        functools.partial(_spmm_body, nch, rpt, chunk),
        out_type=jax.ShapeDtypeStruct((_NC, npad, width), jnp.float32),
        mesh=mesh,
        compiler_params=pltpu.CompilerParams(use_tc_tiling_on_sc=False),
        scratch_types=[
            pltpu.VMEM((nch, chunk), jnp.int32),
            pltpu.VMEM((nch, chunk), jnp.int32),
            pltpu.VMEM((chunk, width), jnp.float32),
            pltpu.VMEM((chunk, width), jnp.float32),
            pltpu.VMEM((chunk, width), jnp.float32),
            pltpu.VMEM_SHARED((npad, width), jnp.float32),
            sem, sem, sem, sem, sem, sem,
        ],
    )


def _spmm(z, src3, dst3):
    return _make_spmm(z.shape[0], src3.shape[1], z.shape[1],
                      src3.shape[2])(z, src3, dst3)


def _mm_body(x_ref, w_ref, o_ref):
    o_ref[...] = jnp.dot(x_ref[...], w_ref[...],
                         preferred_element_type=jnp.float32)


def _matmul(x, w, blk):
    m, k = x.shape
    _, n = w.shape
    return pl.pallas_call(
        _mm_body,
        grid=(m // blk,),
        in_specs=[
            pl.BlockSpec((blk, k), lambda i: (i, 0)),
            pl.BlockSpec((k, n), lambda i: (0, 0)),
        ],
        out_specs=pl.BlockSpec((blk, n), lambda i: (i, 0)),
        out_shape=jax.ShapeDtypeStruct((m, n), jnp.float32),
    )(x, w)


def _dis_of(dacc_ref):
    d = dacc_ref[0, :, 0:1] + dacc_ref[1, :, 0:1] - 1.0
    return lax.rsqrt(d)


def _scale_body(dacc_ref, y_ref, o_ref):
    o_ref[...] = _dis_of(dacc_ref) * y_ref[...]


def _scale(dacc, y, blk):
    m, k = y.shape
    return pl.pallas_call(
        _scale_body,
        grid=(m // blk,),
        in_specs=[
            pl.BlockSpec((2, blk, 16), lambda i: (0, i, 0)),
            pl.BlockSpec((blk, k), lambda i: (i, 0)),
        ],
        out_specs=pl.BlockSpec((blk, k), lambda i: (i, 0)),
        out_shape=jax.ShapeDtypeStruct((m, k), jnp.float32),
    )(dacc, y)


def _combine_mm_body(dacc_ref, a_ref, z_ref, b_ref, w_ref, o_ref):
    dis = _dis_of(dacc_ref)
    h = dis * (a_ref[0] + a_ref[1] - z_ref[...]) + b_ref[...]
    h = jnp.maximum(h, 0.0)
    o_ref[...] = dis * jnp.dot(h, w_ref[...],
                               preferred_element_type=jnp.float32)


def _combine_matmul(dacc, acc, z, b1, w2p, blk):
    m, k = z.shape
    n = w2p.shape[1]
    return pl.pallas_call(
        _combine_mm_body,
        grid=(m // blk,),
        in_specs=[
            pl.BlockSpec((2, blk, 16), lambda i: (0, i, 0)),
            pl.BlockSpec((2, blk, k), lambda i: (0, i, 0)),
            pl.BlockSpec((blk, k), lambda i: (i, 0)),
            pl.BlockSpec((1, k), lambda i: (0, 0)),
            pl.BlockSpec((k, n), lambda i: (0, 0)),
        ],
        out_specs=pl.BlockSpec((blk, n), lambda i: (i, 0)),
        out_shape=jax.ShapeDtypeStruct((m, n), jnp.float32),
    )(dacc, acc, z, b1, w2p)


def _epilogue_body(dacc_ref, a2_ref, zg_ref, b2_ref, o_ref):
    dis = _dis_of(dacc_ref)
    o16 = dis * (a2_ref[0] + a2_ref[1] - zg_ref[...]) + b2_ref[...]
    o_ref[...] = o16[:, :2]


def _epilogue(dacc, acc2, zg, b2p, n):
    return pl.pallas_call(
        _epilogue_body,
        grid=(1,),
        in_specs=[
            pl.BlockSpec((2, n, 16), lambda i: (0, 0, 0)),
            pl.BlockSpec((2, n, 16), lambda i: (0, 0, 0)),
            pl.BlockSpec((n, 16), lambda i: (0, 0)),
            pl.BlockSpec((1, 16), lambda i: (0, 0)),
        ],
        out_specs=pl.BlockSpec((n, 2), lambda i: (0, 0)),
        out_shape=jax.ShapeDtypeStruct((n, 2), jnp.float32),
    )(dacc, acc2, zg, b2p)


def kernel(x, edge_index, W1, b1, W2, b2):
    n = x.shape[0]
    e = edge_index.shape[1]
    npad = ((n + 2047) // 2048) * 2048
    blk = npad // 5
    ei = edge_index.astype(jnp.int32)

    grain = _NW * 128 * 3
    epad = ((e + grain - 1) // grain) * grain
    fill = n + (jnp.arange(epad - e, dtype=jnp.int32) % (npad - n))
    srcp = jnp.concatenate([ei[0], fill])
    dstp = jnp.concatenate([ei[1], fill])
    srcw, dstw = (a.reshape(_NW, -1, 64) for a in (srcp, dstp))
    srcn, dstn = (a.reshape(_NW, -1, 128) for a in (srcp, dstp))

    ones16 = jnp.ones((npad, 16), jnp.float32)
    dacc = _spmm(ones16, srcn, dstn)
    xp = jnp.pad(x, ((0, npad - n), (0, 0)))
    y = _matmul(xp, W1, blk)

    z = _scale(dacc, y, blk)
    acc = _spmm(z, srcw, dstw)
    w2p16 = jnp.pad(W2, ((0, 0), (0, 16 - W2.shape[1])))
    zg = _combine_matmul(dacc, acc, z, b1[None, :], w2p16, blk)

    acc2 = _spmm(zg, srcn, dstn)
    b2p = jnp.pad(b2, (0, 16 - b2.shape[0]))[None, :]
    return _epilogue(dacc, acc2, zg, b2p, n)


# speedup vs baseline: 40.4282x; 1.1533x over previous
import functools

import jax
import jax.numpy as jnp
from jax import lax
from jax.experimental import pallas as pl
from jax.experimental.pallas import tpu as pltpu
from jax.experimental.pallas import tpu_sc as plsc

FDIM = 128

_NC = 2
_NS = 16
_NW = _NC * _NS

_SC_PARAMS = pltpu.CompilerParams(use_tc_tiling_on_sc=False,
                                  needs_layout_passes=False)
_MESH = dict(core_axis_name="c", subcore_axis_name="s")


def _spmm_body(nch, rpt, chunk, eper, z_hbm, srcp, dstp, out_hbm,
               src_v, dst_v, b0, b1, b2, acc,
               g0, g1, g2, s0, s1, s2):
    c = lax.axis_index("c")
    s = lax.axis_index("s")
    wid = c * _NS + s
    bufs = (b0, b1, b2)
    gsem = (g0, g1, g2)
    ssem = (s0, s1, s2)

    pltpu.sync_copy(srcp.at[pl.ds(wid * eper, eper)], src_v)
    pltpu.sync_copy(dstp.at[pl.ds(wid * eper, eper)], dst_v)

    base = s * rpt
    pltpu.sync_copy(z_hbm.at[pl.ds(base, rpt)], acc.at[pl.ds(base, rpt)])
    plsc.subcore_barrier()

    def idx(ref, j):
        return ref.at[pl.ds(j * chunk, chunk)]

    def gather(j, k):
        return pltpu.async_copy(z_hbm.at[idx(src_v, j)], bufs[k], gsem[k])

    def gather_wait(j, k):
        pltpu.make_async_copy(z_hbm.at[idx(src_v, j)], bufs[k],
                              gsem[k]).wait()

    def scatter(j, k):
        return pltpu.async_copy(bufs[k], acc.at[idx(dst_v, j)], ssem[k],
                                add=True)

    def scatter_wait(j, k):
        pltpu.make_async_copy(bufs[k], acc.at[idx(dst_v, j)], ssem[k]).wait()

    gather(0, 0)
    gather(1, 1)
    gather_wait(0, 0)
    scatter(0, 0)
    gather(2, 2)

    def body(i, carry):
        for k, j in ((1, 3 * i + 1), (2, 3 * i + 2), (0, 3 * i + 3)):
            kn = (k + 2) % 3
            gather_wait(j, k)
            scatter(j, k)
            scatter_wait(j - 1, kn)
            gather(j + 2, kn)
        return carry

    lax.fori_loop(0, (nch - 3) // 3, body, 0)
    for j, k in ((nch - 2, 1), (nch - 1, 2)):
        gather_wait(j, k)
        scatter(j, k)
    for j, k in ((nch - 3, 0), (nch - 2, 1), (nch - 1, 2)):
        scatter_wait(j, k)
    plsc.subcore_barrier()

    pltpu.sync_copy(acc.at[pl.ds(base, rpt)], out_hbm.at[c, pl.ds(base, rpt)])


@functools.lru_cache(maxsize=None)
def _make_spmm(npad, eper, chunk):
    rpt = npad // _NS
    nch = eper // chunk
    sem = pltpu.SemaphoreType.DMA
    return pl.kernel(
        functools.partial(_spmm_body, nch, rpt, chunk, eper),
        out_type=jax.ShapeDtypeStruct((_NC, npad, FDIM), jnp.float32),
        mesh=plsc.VectorSubcoreMesh(**_MESH),
        compiler_params=_SC_PARAMS,
        scratch_types=[
            pltpu.VMEM((eper,), jnp.int32),
            pltpu.VMEM((eper,), jnp.int32),
            pltpu.VMEM((chunk, FDIM), jnp.float32),
            pltpu.VMEM((chunk, FDIM), jnp.float32),
            pltpu.VMEM((chunk, FDIM), jnp.float32),
            pltpu.VMEM_SHARED((npad, FDIM), jnp.float32),
            sem, sem, sem, sem, sem, sem,
        ],
    )


def _publish_reduce(s, c, part_v, shared, tmp, red, out_hbm, seg):
    pltpu.sync_copy(part_v, shared.at[s])
    plsc.subcore_barrier()
    sl = pl.ds(s * seg, seg)
    pltpu.sync_copy(shared.at[0, sl], red)

    def rw(w, carry):
        pltpu.sync_copy(shared.at[w, sl], tmp)

        def rr(i, c2):
            v = pl.ds(i * 16, 16)
            red[v] = red[v] + tmp[v]
            return c2

        lax.fori_loop(0, seg // 16, rr, 0)
        return carry

    lax.fori_loop(1, _NS, rw, 0)
    pltpu.sync_copy(red, out_hbm.at[c, pl.ds(s * seg, seg)])


def _deg_body(eper, npad, dstp, out_hbm, dst_v, hist, tmp, red, shared):
    c = lax.axis_index("c")
    s = lax.axis_index("s")
    wid = c * _NS + s
    pltpu.sync_copy(dstp.at[pl.ds(wid * eper, eper)], dst_v)

    def z(i, carry):
        hist[pl.ds(i * 16, 16)] = jnp.zeros((16,), jnp.float32)
        return carry

    lax.fori_loop(0, npad // 16, z, 0)
    ones = jnp.ones((16,), jnp.float32)

    def b(i, carry):
        idx = dst_v[pl.ds(i * 16, 16)]
        plsc.addupdate_scatter(hist, [idx], ones)
        return carry

    lax.fori_loop(0, eper // 16, b, 0)
    _publish_reduce(s, c, hist, shared, tmp, red, out_hbm, npad // _NS)


@functools.lru_cache(maxsize=None)
def _make_deg(npad, eper):
    seg = npad // _NS
    return pl.kernel(
        functools.partial(_deg_body, eper, npad),
        out_type=jax.ShapeDtypeStruct((_NC, npad), jnp.float32),
        mesh=plsc.VectorSubcoreMesh(**_MESH),
        compiler_params=_SC_PARAMS,
        scratch_types=[
            pltpu.VMEM((eper,), jnp.int32),
            pltpu.VMEM((npad,), jnp.float32),
            pltpu.VMEM((seg,), jnp.float32),
            pltpu.VMEM((seg,), jnp.float32),
            pltpu.VMEM_SHARED((_NS, npad), jnp.float32),
        ],
    )


def _l2_body(eper, npad, zg_hbm, srcp, dstp, out_hbm,
             src_v, dst_v, zg_v, hist, tmp, red, shared):
    c = lax.axis_index("c")
    s = lax.axis_index("s")
    wid = c * _NS + s
    pltpu.sync_copy(srcp.at[pl.ds(wid * eper, eper)], src_v)
    pltpu.sync_copy(dstp.at[pl.ds(wid * eper, eper)], dst_v)
    pltpu.sync_copy(zg_hbm, zg_v)

    def z(i, carry):
        hist[pl.ds(i * 16, 16)] = jnp.zeros((16,), jnp.float32)
        return carry

    lax.fori_loop(0, (npad * 2) // 16, z, 0)

    def b(i, carry):
        si = src_v[pl.ds(i * 16, 16)] * 2
        di = dst_v[pl.ds(i * 16, 16)] * 2
        v0 = plsc.load_gather(zg_v, [si])
        v1 = plsc.load_gather(zg_v, [si + 1])
        plsc.addupdate_scatter(hist, [di], v0)
        plsc.addupdate_scatter(hist, [di + 1], v1)
        return carry

    lax.fori_loop(0, eper // 16, b, 0)
    _publish_reduce(s, c, hist, shared, tmp, red, out_hbm,
                    (npad * 2) // _NS)


@functools.lru_cache(maxsize=None)
def _make_l2(npad, eper):
    seg = (npad * 2) // _NS
    return pl.kernel(
        functools.partial(_l2_body, eper, npad),
        out_type=jax.ShapeDtypeStruct((_NC, npad * 2), jnp.float32),
        mesh=plsc.VectorSubcoreMesh(**_MESH),
        compiler_params=_SC_PARAMS,
        scratch_types=[
            pltpu.VMEM((eper,), jnp.int32),
            pltpu.VMEM((eper,), jnp.int32),
            pltpu.VMEM((npad * 2,), jnp.float32),
            pltpu.VMEM((npad * 2,), jnp.float32),
            pltpu.VMEM((seg,), jnp.float32),
            pltpu.VMEM((seg,), jnp.float32),
            pltpu.VMEM_SHARED((_NS, npad * 2), jnp.float32),
        ],
    )


def _mm_body(x_ref, w_ref, o_ref):
    o_ref[...] = jnp.dot(x_ref[...], w_ref[...],
                         preferred_element_type=jnp.float32)


def _matmul(x, w, blk):
    m, k = x.shape
    _, n = w.shape
    return pl.pallas_call(
        _mm_body,
        grid=(m // blk,),
        in_specs=[
            pl.BlockSpec((blk, k), lambda i: (i, 0)),
            pl.BlockSpec((k, n), lambda i: (0, 0)),
        ],
        out_specs=pl.BlockSpec((blk, n), lambda i: (i, 0)),
        out_shape=jax.ShapeDtypeStruct((m, n), jnp.float32),
    )(x, w)


def _scale_body(d_ref, y_ref, o_ref):
    o_ref[...] = lax.rsqrt(d_ref[...]) * y_ref[...]


def _scale(degc, y, blk):
    m, k = y.shape
    return pl.pallas_call(
        _scale_body,
        grid=(m // blk,),
        in_specs=[
            pl.BlockSpec((blk, 1), lambda i: (i, 0)),
            pl.BlockSpec((blk, k), lambda i: (i, 0)),
        ],
        out_specs=pl.BlockSpec((blk, k), lambda i: (i, 0)),
        out_shape=jax.ShapeDtypeStruct((m, k), jnp.float32),
    )(degc, y)


def _combine_mm_body(d_ref, a_ref, z_ref, b_ref, w_ref, o_ref):
    dis = lax.rsqrt(d_ref[...])
    h = dis * (a_ref[0] + a_ref[1] - z_ref[...]) + b_ref[...]
    h = jnp.maximum(h, 0.0)
    o_ref[...] = dis * jnp.dot(h, w_ref[...],
                               preferred_element_type=jnp.float32)


def _combine_matmul(degc, acc, z, b1, w2, blk):
    m, k = z.shape
    n = w2.shape[1]
    return pl.pallas_call(
        _combine_mm_body,
        grid=(m // blk,),
        in_specs=[
            pl.BlockSpec((blk, 1), lambda i: (i, 0)),
            pl.BlockSpec((2, blk, k), lambda i: (0, i, 0)),
            pl.BlockSpec((blk, k), lambda i: (i, 0)),
            pl.BlockSpec((1, k), lambda i: (0, 0)),
            pl.BlockSpec((k, n), lambda i: (0, 0)),
        ],
        out_specs=pl.BlockSpec((blk, n), lambda i: (i, 0)),
        out_shape=jax.ShapeDtypeStruct((m, n), jnp.float32),
    )(degc, acc, z, b1, w2)


def _epilogue_body(d_ref, a2_ref, zg_ref, b2_ref, o_ref):
    dis = lax.rsqrt(d_ref[...])
    o_ref[...] = dis * (a2_ref[0] + a2_ref[1] + zg_ref[...]) + b2_ref[...]


def _epilogue(degc, acc2, zg2, b2r, n):
    return pl.pallas_call(
        _epilogue_body,
        grid=(1,),
        in_specs=[
            pl.BlockSpec((n, 1), lambda i: (0, 0)),
            pl.BlockSpec((2, n, 2), lambda i: (0, 0, 0)),
            pl.BlockSpec((n, 2), lambda i: (0, 0)),
            pl.BlockSpec((1, 2), lambda i: (0, 0)),
        ],
        out_specs=pl.BlockSpec((n, 2), lambda i: (0, 0)),
        out_shape=jax.ShapeDtypeStruct((n, 2), jnp.float32),
    )(degc, acc2, zg2, b2r)


def kernel(x, edge_index, W1, b1, W2, b2):
    n = x.shape[0]
    e = edge_index.shape[1]
    npad = ((n + 2047) // 2048) * 2048
    blk = npad // 5
    ei = edge_index.astype(jnp.int32)

    grain = _NW * 128 * 3
    epad = ((e + grain - 1) // grain) * grain
    eper = epad // _NW
    fill = n + (jnp.arange(epad - e, dtype=jnp.int32) % (npad - n))
    srcp = jnp.concatenate([ei[0], fill])
    dstp = jnp.concatenate([ei[1], fill])

    dacc = _make_deg(npad, eper)(dstp)
    xp = jnp.pad(x, ((0, npad - n), (0, 0)))
    y = _matmul(xp, W1, blk)
    degc = (dacc[0] + dacc[1] + 1.0)[:, None]

    z = _scale(degc, y, blk)
    acc = _make_spmm(npad, eper, 64)(z, srcp, dstp)
    zg2 = _combine_matmul(degc, acc, z, b1[None, :], W2, blk)

    s2 = _make_l2(npad, eper)(zg2.reshape(-1), srcp, dstp)
    return _epilogue(degc, s2.reshape(_NC, npad, 2), zg2,
                     b2[None, :], n)
